# baseline (speedup 1.0000x reference)
import jax
import jax.numpy as jnp
from jax.experimental import pallas as pl


def _probe(x_ref, inc_ref, out_ref):
    out_ref[...] = x_ref[...] * 2.0 + inc_ref[0]


def kernel(node_features, incidence_matrix, W, b, epsilon):
    N, D = node_features.shape
    inc1 = incidence_matrix.reshape(-1)
    return pl.pallas_call(
        _probe,
        out_shape=jax.ShapeDtypeStruct((N, D), jnp.float32),
    )(node_features, inc1)


# speedup vs baseline: 1.2131x; 1.2131x over previous
import jax
import jax.numpy as jnp
from jax import lax
from jax.experimental import pallas as pl
from jax.experimental.pallas import tpu as pltpu

_K = 8


def _dot_t(a, b):
    return lax.dot_general(a, b, (((0,), (0,)), ((), ())),
                           preferred_element_type=jnp.float32)


def _hgmp_kernel(x_ref, inc_hbm, w_ref, b_ref, eps_ref, out_ref, inc_sc, sems):
    n = x_ref.shape[0]
    rows = n // _K
    cps = []
    for i in range(_K):
        cp = pltpu.make_async_copy(
            inc_hbm.at[pl.ds(i * rows, rows), :],
            inc_sc.at[pl.ds(i * rows, rows), :],
            sems.at[i])
        cp.start()
        cps.append(cp)

    x = x_ref[...]
    w = w_ref[...]
    xw = lax.dot_general(x, w, (((1,), (1,)), ((), ())),
                         preferred_element_type=jnp.float32)
    xwb = (1.0 + eps_ref[0, 0]) * xw + b_ref[...]

    sums = jnp.zeros((inc_sc.shape[1], x.shape[1]), jnp.float32)
    counts = jnp.zeros((inc_sc.shape[1], 1), jnp.float32)
    ones = jnp.ones((rows, 1), dtype=jnp.float32)
    for i in range(_K):
        cps[i].wait()
        m = (inc_sc[pl.ds(i * rows, rows), :] > 0).astype(jnp.float32)
        sums = sums + _dot_t(m, x[i * rows:(i + 1) * rows, :])
        counts = counts + _dot_t(m, ones)

    vf = sums / jnp.maximum(counts, 1.0)
    vfw = lax.dot_general(vf, w, (((1,), (1,)), ((), ())),
                          preferred_element_type=jnp.float32)

    mask = (inc_sc[...] > 0).astype(jnp.float32)
    y = xwb + jnp.dot(mask, vfw, preferred_element_type=jnp.float32)
    out_ref[...] = jnp.where(y > 0, y, 0.2 * y)


def kernel(node_features, incidence_matrix, W, b, epsilon):
    N, D = node_features.shape
    V = incidence_matrix.shape[1]
    b2 = b.reshape(1, D)
    eps2 = epsilon.reshape(1, 1)
    return pl.pallas_call(
        _hgmp_kernel,
        in_specs=[
            pl.BlockSpec((N, D), lambda: (0, 0)),
            pl.BlockSpec(memory_space=pl.ANY),
            pl.BlockSpec((D, D), lambda: (0, 0)),
            pl.BlockSpec((1, D), lambda: (0, 0)),
            pl.BlockSpec((1, 1), lambda: (0, 0)),
        ],
        out_specs=pl.BlockSpec((N, D), lambda: (0, 0)),
        out_shape=jax.ShapeDtypeStruct((N, D), jnp.float32),
        scratch_shapes=[
            pltpu.VMEM((N, V), jnp.float32),
            pltpu.SemaphoreType.DMA((_K,)),
        ],
    )(node_features, incidence_matrix, W, b2, eps2)
